# baseline (speedup 1.0000x reference)
import functools

import jax
import jax.numpy as jnp
from jax import lax
from jax.experimental import pallas as pl
from jax.experimental.pallas import tpu as pltpu
from jax.experimental.pallas import tpu_sc as plsc

VOCAB = 1000000
EMBED = 32

NC = 2
NS = 16
NW = NC * NS
B_TOTAL = 4096 * 200
ROWS_PW = B_TOTAL // NW
IPS = 128
K = 5
RPB = K * IPS
NBLK = ROWS_PW // RPB
NROWCH = ROWS_PW // IPS


@functools.partial(
    pl.kernel,
    out_type=jax.ShapeDtypeStruct((B_TOTAL, EMBED), jnp.float32),
    mesh=plsc.VectorSubcoreMesh(core_axis_name="c", subcore_axis_name="s"),
    scratch_types=[
        pltpu.VMEM((NROWCH, IPS), jnp.int32),
        pltpu.VMEM((2, RPB, EMBED), jnp.float32),
        pltpu.SemaphoreType.DMA,
        pltpu.SemaphoreType.DMA,
    ],
    compiler_params=pltpu.CompilerParams(use_tc_tiling_on_sc=False),
)
def _emb_lookup(x_hbm, table_hbm, out_hbm, idx_v, rows_v, gsem, osem):
    wid = lax.axis_index("s") * NC + lax.axis_index("c")
    pltpu.sync_copy(x_hbm.at[wid], idx_v)
    out_base = wid * ROWS_PW

    def pair_body(t, carry):
        for b in range(2):
            blk = 2 * t + b

            @pl.when(blk >= 2)
            def _():
                pltpu.make_async_copy(
                    rows_v.at[b],
                    out_hbm.at[pl.ds(out_base + (blk - 2) * RPB, RPB)],
                    osem,
                ).wait()

            descs = [
                pltpu.async_copy(
                    table_hbm.at[idx_v.at[blk * K + j]],
                    rows_v.at[b, pl.ds(j * IPS, IPS)],
                    gsem,
                )
                for j in range(K)
            ]
            for d in descs:
                d.wait()
            pltpu.async_copy(
                rows_v.at[b],
                out_hbm.at[pl.ds(out_base + blk * RPB, RPB)],
                osem,
            )
        return carry

    lax.fori_loop(0, NBLK // 2, pair_body, 0)
    for b in range(2):
        pltpu.make_async_copy(
            rows_v.at[b],
            out_hbm.at[pl.ds(out_base + (NBLK - 2 + b) * RPB, RPB)],
            osem,
        ).wait()


def kernel(x, table):
    x_r = x.reshape(NW, NROWCH, IPS).astype(jnp.int32)
    out = _emb_lookup(x_r, table)
    return out.reshape(x.shape[0], x.shape[1], EMBED)
